# Initial kernel scaffold; baseline (speedup 1.0000x reference)
import jax
import jax.numpy as jnp
from jax.experimental import pallas as pl


def kernel(shape_ids, color_ids, edge_index, batch, shape_emb, color_emb, W1, b1, W2, b2, Wc, bc):
    raise NotImplementedError("write your pallas kernel here")


# speedup vs baseline: 1.0000x; 1.0000x over previous
import jax
import jax.numpy as jnp
from jax.experimental import pallas as pl


def _gcn_conv(x, edge_index, W, b):
    n = x.shape[0]
    loop = jnp.arange(n, dtype=edge_index.dtype)
    src = jnp.concatenate([edge_index[0], loop])
    dst = jnp.concatenate([edge_index[1], loop])
    deg = jnp.zeros((n,), dtype=x.dtype).at[dst].add(1.0)
    deg_inv_sqrt = jnp.where(deg > 0, deg ** -0.5, 0.0)
    norm = deg_inv_sqrt[src] * deg_inv_sqrt[dst]
    h = x @ W
    msg = h[src] * norm[:, None]
    out = jnp.zeros((n, h.shape[1]), dtype=x.dtype).at[dst].add(msg)
    return out + b


def _final_mm_kernel(p_ref, w_ref, b_ref, o_ref):
    o_ref[...] = p_ref[...] @ w_ref[...] + b_ref[...]


def kernel(shape_ids, color_ids, edge_index, batch, shape_emb, color_emb, W1, b1, W2, b2, Wc, bc):
    x = jnp.concatenate([jnp.take(shape_emb, shape_ids, axis=0), jnp.take(color_emb, color_ids, axis=0)], axis=1)
    x = jax.nn.relu(_gcn_conv(x, edge_index, W1, b1))
    x = jax.nn.relu(_gcn_conv(x, edge_index, W2, b2))
    B = 128
    sums = jax.ops.segment_sum(x, batch, num_segments=B)
    counts = jax.ops.segment_sum(jnp.ones((x.shape[0],), dtype=x.dtype), batch, num_segments=B)
    pooled = sums / jnp.maximum(counts, 1.0)[:, None]
    out = pl.pallas_call(
        _final_mm_kernel,
        out_shape=jax.ShapeDtypeStruct((B, Wc.shape[1]), jnp.float32),
    )(pooled, Wc, bc)
    return out


# speedup vs baseline: 24.3317x; 24.3310x over previous
import functools

import jax
import jax.numpy as jnp
from jax import lax
from jax.experimental import pallas as pl
from jax.experimental.pallas import tpu as pltpu
from jax.experimental.pallas import tpu_sc as plsc

N = 100000
E = 1600000
B = 128
HIDDEN = 32
NUM_CLASSES = 2

NC, NS = 2, 16
NW = NC * NS
N_PAD = 102400
NODES_PER_TILE = N_PAD // NW
NROWCH = NODES_PER_TILE // 128
E_PAD = 1605632
ER = E_PAD // 128
P1_ROWS = ER // NW
P1_CH = P1_ROWS // 8
CV_ROWS = ER // NS
CV_K = 4
CV_SUP = CV_ROWS // CV_K
HALF = 50048
NHI = 2 * HALF
ACCROWS = 50176
ACC_STRIPE = ACCROWS // NS
WB_STRIPE = HALF // NS
DEGROWS = 104448
DEG_STRIPE = DEGROWS // NS
RB = 512
NB = N_PAD // RB

_mesh = plsc.VectorSubcoreMesh(core_axis_name="c", subcore_axis_name="s")


def _prep_body(se_ref, ce_ref, w1_ref, ct_ref):
    a = jnp.dot(se_ref[...], w1_ref[0:8, :], preferred_element_type=jnp.float32)
    b = jnp.dot(ce_ref[...], w1_ref[8:12, :], preferred_element_type=jnp.float32)
    row = lax.broadcasted_iota(jnp.int32, (260, 26), 0)
    ohs = (row // 10 == lax.broadcasted_iota(jnp.int32, (260, 26), 1)).astype(jnp.float32)
    rowc = lax.broadcasted_iota(jnp.int32, (260, 10), 0)
    ohc = (rowc % 10 == lax.broadcasted_iota(jnp.int32, (260, 10), 1)).astype(jnp.float32)
    ct_ref[...] = (jnp.dot(ohs, a, preferred_element_type=jnp.float32)
                   + jnp.dot(ohc, b, preferred_element_type=jnp.float32))


def _prep(shape_emb, color_emb, W1):
    return pl.pallas_call(
        _prep_body,
        out_shape=jax.ShapeDtypeStruct((260, HIDDEN), jnp.float32),
    )(shape_emb, color_emb, W1)


def _p1_body(sid_hbm, col_hbm, dst_hbm, ct_hbm,
             h1_hbm, degpart_hbm, idx_hbm,
             degacc, sbuf, cbuf, cidbuf, rowbuf, dstbuf, lobuf, hibuf,
             onesbuf, zbuf,
             nsem, ngsem, hwsem, dsem, degsem, wsem):
    c = lax.axis_index("c")
    s = lax.axis_index("s")
    w = c * NS + s

    zeros16 = jnp.zeros((16,), jnp.float32)
    ones16 = jnp.ones((16,), jnp.float32)

    @pl.loop(0, 8)
    def _init(k):
        onesbuf[pl.ds(k * 16, 16)] = ones16

    @pl.loop(0, DEG_STRIPE // 16)
    def _initz(k):
        zbuf[pl.ds(k * 16, 16)] = zeros16

    pltpu.sync_copy(zbuf, degacc.at[pl.ds(s * DEG_STRIPE, DEG_STRIPE)])

    nrow0 = w * NROWCH

    def _fire_idcopy(j, d):
        pltpu.async_copy(sid_hbm.at[nrow0 + j], sbuf.at[d], nsem)
        pltpu.async_copy(col_hbm.at[nrow0 + j], cbuf.at[d], nsem)

    _fire_idcopy(0, 0)

    @pl.loop(0, NROWCH)
    def _node_chunk(j):
        d = lax.rem(j, 2)
        dn = lax.rem(j + 1, 2)
        pltpu.make_async_copy(sid_hbm.at[nrow0 + j], sbuf.at[d], nsem).wait()
        pltpu.make_async_copy(col_hbm.at[nrow0 + j], cbuf.at[d], nsem).wait()

        @pl.when(j + 1 < NROWCH)
        def _():
            _fire_idcopy(j + 1, dn)

        @pl.when(j > 1)
        def _():
            pltpu.make_async_copy(
                rowbuf.at[d], h1_hbm.at[pl.ds(0, 128)], hwsem).wait()

        for k in range(8):
            sl = pl.ds(k * 16, 16)
            cidbuf[d, sl] = sbuf[d, sl] * 10 + cbuf[d, sl]
        pltpu.async_copy(ct_hbm.at[cidbuf.at[d]], rowbuf.at[d], ngsem).wait()
        pltpu.async_copy(
            rowbuf.at[d],
            h1_hbm.at[pl.ds(w * NODES_PER_TILE + j * 128, 128)], hwsem)

    pltpu.make_async_copy(rowbuf.at[0], h1_hbm.at[pl.ds(0, 128)], hwsem).wait()
    pltpu.make_async_copy(rowbuf.at[0], h1_hbm.at[pl.ds(0, 128)], hwsem).wait()

    plsc.subcore_barrier()

    erow0 = w * P1_ROWS

    def _fire_dst(k, d):
        pltpu.async_copy(dst_hbm.at[pl.ds(erow0 + k * 8, 8)], dstbuf.at[d], dsem)

    _fire_dst(0, 0)

    @pl.loop(0, P1_CH)
    def _edge_chunk(k):
        d = lax.rem(k, 2)
        dn = lax.rem(k + 1, 2)
        pltpu.make_async_copy(
            dst_hbm.at[pl.ds(erow0, 8)], dstbuf.at[d], dsem).wait()

        @pl.when(k > 0)
        def _():
            for j2 in range(8):
                pltpu.make_async_copy(
                    onesbuf, degacc.at[dstbuf.at[dn, j2]], degsem).wait()

        @pl.when(k + 1 < P1_CH)
        def _():
            _fire_dst(k + 1, dn)

        @pl.when(k > 1)
        def _():
            pltpu.make_async_copy(
                lobuf.at[d], idx_hbm.at[0, pl.ds(0, 8)], wsem).wait()
            pltpu.make_async_copy(
                hibuf.at[d], idx_hbm.at[1, pl.ds(0, 8)], wsem).wait()

        for j2 in range(8):
            for kk in range(8):
                sl = pl.ds(kk * 16, 16)
                dv = dstbuf[d, j2, sl]
                trash = HALF + (dv & 127)
                lobuf[d, j2, sl] = jnp.where(dv < HALF, dv, trash)
                inhi = (dv >= HALF) & (dv < NHI)
                hibuf[d, j2, sl] = jnp.where(inhi, dv - HALF, trash)

        for j2 in range(8):
            pltpu.async_copy(onesbuf, degacc.at[dstbuf.at[d, j2]], degsem,
                             add=True)
        pltpu.async_copy(lobuf.at[d],
                         idx_hbm.at[0, pl.ds(erow0 + k * 8, 8)], wsem)
        pltpu.async_copy(hibuf.at[d],
                         idx_hbm.at[1, pl.ds(erow0 + k * 8, 8)], wsem)

    for j2 in range(8):
        pltpu.make_async_copy(
            onesbuf, degacc.at[dstbuf.at[0, j2]], degsem).wait()
    for _ in range(2):
        pltpu.make_async_copy(
            lobuf.at[0], idx_hbm.at[0, pl.ds(0, 8)], wsem).wait()
        pltpu.make_async_copy(
            hibuf.at[0], idx_hbm.at[1, pl.ds(0, 8)], wsem).wait()

    plsc.subcore_barrier()

    pltpu.sync_copy(degacc.at[pl.ds(s * (N_PAD // NS), N_PAD // NS)],
                    degpart_hbm.at[c, pl.ds(s * (N_PAD // NS), N_PAD // NS)])


def _p1(sid2d, col2d, dst2d, ctable):
    f = pl.kernel(
        _p1_body,
        out_type=[
            jax.ShapeDtypeStruct((N_PAD, HIDDEN), jnp.float32),
            jax.ShapeDtypeStruct((NC, N_PAD), jnp.float32),
            jax.ShapeDtypeStruct((NC, ER, 128), jnp.int32),
        ],
        mesh=_mesh,
        compiler_params=pltpu.CompilerParams(use_tc_tiling_on_sc=False),
        scratch_types=[
            pltpu.VMEM_SHARED((DEGROWS,), jnp.float32),
            pltpu.VMEM((2, 128), jnp.int32),
            pltpu.VMEM((2, 128), jnp.int32),
            pltpu.VMEM((2, 128), jnp.int32),
            pltpu.VMEM((2, 128, HIDDEN), jnp.float32),
            pltpu.VMEM((2, 8, 128), jnp.int32),
            pltpu.VMEM((2, 8, 128), jnp.int32),
            pltpu.VMEM((2, 8, 128), jnp.int32),
            pltpu.VMEM((128,), jnp.float32),
            pltpu.VMEM((DEG_STRIPE,), jnp.float32),
            pltpu.SemaphoreType.DMA,
            pltpu.SemaphoreType.DMA,
            pltpu.SemaphoreType.DMA,
            pltpu.SemaphoreType.DMA,
            pltpu.SemaphoreType.DMA,
            pltpu.SemaphoreType.DMA,
        ],
    )
    return f(sid2d, col2d, dst2d, ctable)


def _conv_body(hp_hbm, src_hbm, idx_hbm,
               accout_hbm,
               acc, srcidx, sidx, rowbuf,
               isem, gsem, ssem, zsem):
    c = lax.axis_index("c")
    s = lax.axis_index("s")

    zeros16 = jnp.zeros((16,), jnp.float32)

    @pl.loop(0, 128)
    def _initz(i):
        for b in range(CV_K):
            rowbuf[b, i, pl.ds(0, 16)] = zeros16
            rowbuf[b, i, pl.ds(16, 16)] = zeros16

    for z in range(24):
        pltpu.async_copy(
            rowbuf.at[0], acc.at[pl.ds(s * ACC_STRIPE + z * 128, 128)], zsem)
    pltpu.async_copy(rowbuf.at[0, pl.ds(0, 64)],
                     acc.at[pl.ds(s * ACC_STRIPE + 24 * 128, 64)], zsem)
    for z in range(24):
        pltpu.make_async_copy(rowbuf.at[0], acc.at[pl.ds(0, 128)], zsem).wait()
    pltpu.make_async_copy(rowbuf.at[0, pl.ds(0, 64)],
                          acc.at[pl.ds(0, 64)], zsem).wait()
    plsc.subcore_barrier()

    row0 = s * CV_ROWS

    def _fire_idx(sup, d):
        for b in range(CV_K):
            r = row0 + sup * CV_K + b
            pltpu.async_copy(src_hbm.at[r], srcidx.at[d, b], isem)
            pltpu.async_copy(idx_hbm.at[c, r], sidx.at[d, b], isem)

    _fire_idx(0, 0)

    @pl.loop(0, CV_SUP)
    def _super(sp):
        d = lax.rem(sp, 2)
        dn = lax.rem(sp + 1, 2)
        @pl.when(sp > 0)
        def _():
            for b in range(CV_K):
                pltpu.make_async_copy(
                    rowbuf.at[b], acc.at[sidx.at[dn, b]], ssem).wait()
        for b in range(CV_K):
            pltpu.make_async_copy(src_hbm.at[0], srcidx.at[d, b], isem).wait()
            pltpu.make_async_copy(src_hbm.at[0], sidx.at[d, b], isem).wait()
        gd = [pltpu.async_copy(hp_hbm.at[srcidx.at[d, b]], rowbuf.at[b],
                               gsem) for b in range(CV_K)]
        @pl.when(sp + 1 < CV_SUP)
        def _():
            _fire_idx(sp + 1, dn)
        for g in gd:
            g.wait()
        for b in range(CV_K):
            pltpu.async_copy(rowbuf.at[b], acc.at[sidx.at[d, b]], ssem,
                             add=True)

    for b in range(CV_K):
        pltpu.make_async_copy(
            rowbuf.at[b], acc.at[sidx.at[0, b]], ssem).wait()

    plsc.subcore_barrier()
    pltpu.sync_copy(
        acc.at[pl.ds(s * WB_STRIPE, WB_STRIPE)],
        accout_hbm.at[pl.ds(c * HALF + s * WB_STRIPE, WB_STRIPE)])


def _conv(hp, src2d, idxarr):
    f = pl.kernel(
        _conv_body,
        out_type=jax.ShapeDtypeStruct((N_PAD, HIDDEN), jnp.float32),
        mesh=_mesh,
        compiler_params=pltpu.CompilerParams(use_tc_tiling_on_sc=False),
        scratch_types=[
            pltpu.VMEM_SHARED((ACCROWS, HIDDEN), jnp.float32),
            pltpu.VMEM((2, CV_K, 128), jnp.int32),
            pltpu.VMEM((2, CV_K, 128), jnp.int32),
            pltpu.VMEM((CV_K, 128, HIDDEN), jnp.float32),
            pltpu.SemaphoreType.DMA,
            pltpu.SemaphoreType.DMA,
            pltpu.SemaphoreType.DMA,
            pltpu.SemaphoreType.DMA,
        ],
    )
    return f(hp, src2d, idxarr)


def _scale1_body(d0_ref, d1_ref, h1_ref, dinv_ref, h1p_ref):
    deg = d0_ref[0, 0] + d1_ref[0, 0] + 1.0
    dv = lax.rsqrt(deg)
    dinv_ref[0, 0] = dv
    h1p_ref[0] = h1_ref[0] * dv[:, None]


def _scale1(d0, d1, h1):
    return pl.pallas_call(
        _scale1_body,
        grid=(NB,),
        in_specs=[
            pl.BlockSpec((1, 1, RB), lambda i: (i, 0, 0)),
            pl.BlockSpec((1, 1, RB), lambda i: (i, 0, 0)),
            pl.BlockSpec((1, RB, HIDDEN), lambda i: (i, 0, 0)),
        ],
        out_specs=[
            pl.BlockSpec((1, 1, RB), lambda i: (i, 0, 0)),
            pl.BlockSpec((1, RB, HIDDEN), lambda i: (i, 0, 0)),
        ],
        out_shape=[
            jax.ShapeDtypeStruct((NB, 1, RB), jnp.float32),
            jax.ShapeDtypeStruct((NB, RB, HIDDEN), jnp.float32),
        ],
    )(d0, d1, h1)


def _scale2_body(acc_ref, h1p_ref, dinv_ref, b1_ref, w2_ref, h2p_ref):
    dv = dinv_ref[0, 0]
    out1 = jnp.maximum((acc_ref[0] + h1p_ref[0]) * dv[:, None] + b1_ref[0],
                       0.0)
    h2 = jnp.dot(out1, w2_ref[...], preferred_element_type=jnp.float32)
    h2p_ref[0] = h2 * dv[:, None]


def _scale2(acc1, h1p, dinv, b1, W2):
    return pl.pallas_call(
        _scale2_body,
        grid=(NB,),
        in_specs=[
            pl.BlockSpec((1, RB, HIDDEN), lambda i: (i, 0, 0)),
            pl.BlockSpec((1, RB, HIDDEN), lambda i: (i, 0, 0)),
            pl.BlockSpec((1, 1, RB), lambda i: (i, 0, 0)),
            pl.BlockSpec((1, HIDDEN), lambda i: (0, 0)),
            pl.BlockSpec((HIDDEN, HIDDEN), lambda i: (0, 0)),
        ],
        out_specs=pl.BlockSpec((1, RB, HIDDEN), lambda i: (i, 0, 0)),
        out_shape=jax.ShapeDtypeStruct((NB, RB, HIDDEN), jnp.float32),
    )(acc1, h1p, dinv, b1, W2)


def _pool_body(acc_ref, h2p_ref, dinv_ref, batch_ref, b2_ref, wc_ref, bc_ref,
               out_ref, sums, cnt):
    i = pl.program_id(0)

    @pl.when(i == 0)
    def _():
        sums[...] = jnp.zeros((B, HIDDEN), jnp.float32)
        cnt[...] = jnp.zeros((B, 1), jnp.float32)

    dv = dinv_ref[0, 0]
    out2 = jnp.maximum((acc_ref[0] + h2p_ref[0]) * dv[:, None] + b2_ref[0],
                       0.0)
    bt = batch_ref[0, 0]
    out2 = jnp.where(bt[:, None] >= 0, out2, 0.0)
    oh = (lax.broadcasted_iota(jnp.int32, (B, RB), 0)
          == bt[None, :]).astype(jnp.float32)
    sums[...] += jnp.dot(oh, out2, preferred_element_type=jnp.float32)
    cnt[...] += jnp.sum(oh, axis=1, keepdims=True)

    @pl.when(i == NB - 1)
    def _():
        pooled = sums[...] / jnp.maximum(cnt[...], 1.0)
        out_ref[...] = (jnp.dot(pooled, wc_ref[...],
                                preferred_element_type=jnp.float32)
                        + bc_ref[0])


def _pool(acc2, h2p, dinv, batch2d, b2, Wc, bc):
    return pl.pallas_call(
        _pool_body,
        grid=(NB,),
        in_specs=[
            pl.BlockSpec((1, RB, HIDDEN), lambda i: (i, 0, 0)),
            pl.BlockSpec((1, RB, HIDDEN), lambda i: (i, 0, 0)),
            pl.BlockSpec((1, 1, RB), lambda i: (i, 0, 0)),
            pl.BlockSpec((1, 1, RB), lambda i: (i, 0, 0)),
            pl.BlockSpec((1, HIDDEN), lambda i: (0, 0)),
            pl.BlockSpec((HIDDEN, NUM_CLASSES), lambda i: (0, 0)),
            pl.BlockSpec((1, NUM_CLASSES), lambda i: (0, 0)),
        ],
        out_specs=pl.BlockSpec((B, NUM_CLASSES), lambda i: (0, 0)),
        out_shape=jax.ShapeDtypeStruct((B, NUM_CLASSES), jnp.float32),
        scratch_shapes=[
            pltpu.VMEM((B, HIDDEN), jnp.float32),
            pltpu.VMEM((B, 1), jnp.float32),
        ],
    )(acc2, h2p, dinv, batch2d, b2, Wc, bc)


def kernel(shape_ids, color_ids, edge_index, batch, shape_emb, color_emb,
           W1, b1, W2, b2, Wc, bc):
    i32 = jnp.int32
    sid2d = jnp.pad(shape_ids.astype(i32), (0, N_PAD - N)).reshape(N_PAD // 128, 128)
    col2d = jnp.pad(color_ids.astype(i32), (0, N_PAD - N)).reshape(N_PAD // 128, 128)
    src2d = jnp.pad(edge_index[0].astype(i32), (0, E_PAD - E)).reshape(ER, 128)
    dst2d = jnp.pad(edge_index[1].astype(i32), (0, E_PAD - E),
                    constant_values=N_PAD).reshape(ER, 128)
    batch2d = jnp.pad(batch.astype(i32), (0, N_PAD - N),
                      constant_values=-1).reshape(NB, 1, RB)

    ctable = _prep(shape_emb, color_emb, W1)
    h1, degpart, idxarr = _p1(sid2d, col2d, dst2d, ctable)

    d0 = degpart[0].reshape(NB, 1, RB)
    d1 = degpart[1].reshape(NB, 1, RB)
    dinv, h1p3 = _scale1(d0, d1, h1.reshape(NB, RB, HIDDEN))
    h1p = h1p3.reshape(N_PAD, HIDDEN)

    acc1 = _conv(h1p, src2d, idxarr)
    h2p3 = _scale2(acc1.reshape(NB, RB, HIDDEN), h1p3, dinv,
                   b1.reshape(1, HIDDEN), W2)
    acc2 = _conv(h2p3.reshape(N_PAD, HIDDEN), src2d, idxarr)

    return _pool(acc2.reshape(NB, RB, HIDDEN), h2p3, dinv, batch2d,
                 b2.reshape(1, HIDDEN), Wc, bc.reshape(1, NUM_CLASSES))


# speedup vs baseline: 28.9286x; 1.1889x over previous
import functools

import jax
import jax.numpy as jnp
from jax import lax
from jax.experimental import pallas as pl
from jax.experimental.pallas import tpu as pltpu
from jax.experimental.pallas import tpu_sc as plsc

N = 100000
E = 1600000
B = 128
HIDDEN = 32
NUM_CLASSES = 2

NC, NS = 2, 16
NW = NC * NS
N_PAD = 102400
NODES_PER_TILE = N_PAD // NW
NROWCH = NODES_PER_TILE // 128
E_PAD = 1605632
ER = E_PAD // 128
P1_ROWS = ER // NW
P1_CH = P1_ROWS // 8
CV_ROWS = ER // NS
CV_K = 4
CV_SUP = CV_ROWS // CV_K
HALF = 50048
NHI = 2 * HALF
ACCROWS = 50176
ACC_STRIPE = ACCROWS // NS
WB_STRIPE = HALF // NS
DEGROWS = 104448
DEG_STRIPE = DEGROWS // NS
RB = 4096
NB = N_PAD // RB

_mesh = plsc.VectorSubcoreMesh(core_axis_name="c", subcore_axis_name="s")


def _prep_body(se_ref, ce_ref, w1_ref, ct_ref):
    a = jnp.dot(se_ref[...], w1_ref[0:8, :], preferred_element_type=jnp.float32)
    b = jnp.dot(ce_ref[...], w1_ref[8:12, :], preferred_element_type=jnp.float32)
    row = lax.broadcasted_iota(jnp.int32, (260, 26), 0)
    ohs = (row // 10 == lax.broadcasted_iota(jnp.int32, (260, 26), 1)).astype(jnp.float32)
    rowc = lax.broadcasted_iota(jnp.int32, (260, 10), 0)
    ohc = (rowc % 10 == lax.broadcasted_iota(jnp.int32, (260, 10), 1)).astype(jnp.float32)
    ct_ref[...] = (jnp.dot(ohs, a, preferred_element_type=jnp.float32)
                   + jnp.dot(ohc, b, preferred_element_type=jnp.float32))


def _prep(shape_emb, color_emb, W1):
    return pl.pallas_call(
        _prep_body,
        out_shape=jax.ShapeDtypeStruct((260, HIDDEN), jnp.float32),
    )(shape_emb, color_emb, W1)


def _p1_body(sid_hbm, col_hbm, dst_hbm, ct_hbm,
             h1_hbm, degpart_hbm, idx_hbm,
             degacc, sbuf, cbuf, cidbuf, rowbuf, dstbuf, lobuf, hibuf,
             onesbuf, zbuf,
             nsem, ngsem, hwsem, dsem, degsem, wsem):
    c = lax.axis_index("c")
    s = lax.axis_index("s")
    w = c * NS + s

    zeros16 = jnp.zeros((16,), jnp.float32)
    ones16 = jnp.ones((16,), jnp.float32)

    @pl.loop(0, 8)
    def _init(k):
        onesbuf[pl.ds(k * 16, 16)] = ones16

    @pl.loop(0, DEG_STRIPE // 16)
    def _initz(k):
        zbuf[pl.ds(k * 16, 16)] = zeros16

    pltpu.sync_copy(zbuf, degacc.at[pl.ds(s * DEG_STRIPE, DEG_STRIPE)])

    nrow0 = w * NROWCH

    def _fire_idcopy(j, d):
        pltpu.async_copy(sid_hbm.at[nrow0 + j], sbuf.at[d], nsem)
        pltpu.async_copy(col_hbm.at[nrow0 + j], cbuf.at[d], nsem)

    _fire_idcopy(0, 0)

    @pl.loop(0, NROWCH)
    def _node_chunk(j):
        d = lax.rem(j, 2)
        dn = lax.rem(j + 1, 2)
        pltpu.make_async_copy(sid_hbm.at[nrow0 + j], sbuf.at[d], nsem).wait()
        pltpu.make_async_copy(col_hbm.at[nrow0 + j], cbuf.at[d], nsem).wait()

        @pl.when(j + 1 < NROWCH)
        def _():
            _fire_idcopy(j + 1, dn)

        @pl.when(j > 1)
        def _():
            pltpu.make_async_copy(
                rowbuf.at[d], h1_hbm.at[pl.ds(0, 128)], hwsem).wait()

        for k in range(8):
            sl = pl.ds(k * 16, 16)
            cidbuf[d, sl] = sbuf[d, sl] * 10 + cbuf[d, sl]
        pltpu.async_copy(ct_hbm.at[cidbuf.at[d]], rowbuf.at[d], ngsem).wait()
        pltpu.async_copy(
            rowbuf.at[d],
            h1_hbm.at[pl.ds(w * NODES_PER_TILE + j * 128, 128)], hwsem)

    pltpu.make_async_copy(rowbuf.at[0], h1_hbm.at[pl.ds(0, 128)], hwsem).wait()
    pltpu.make_async_copy(rowbuf.at[0], h1_hbm.at[pl.ds(0, 128)], hwsem).wait()

    plsc.subcore_barrier()

    erow0 = w * P1_ROWS

    def _fire_dst(k, d):
        pltpu.async_copy(dst_hbm.at[pl.ds(erow0 + k * 8, 8)], dstbuf.at[d], dsem)

    _fire_dst(0, 0)

    @pl.loop(0, P1_CH)
    def _edge_chunk(k):
        d = lax.rem(k, 2)
        dn = lax.rem(k + 1, 2)
        pltpu.make_async_copy(
            dst_hbm.at[pl.ds(erow0, 8)], dstbuf.at[d], dsem).wait()

        @pl.when(k > 0)
        def _():
            for j2 in range(8):
                pltpu.make_async_copy(
                    onesbuf, degacc.at[dstbuf.at[dn, j2]], degsem).wait()

        @pl.when(k + 1 < P1_CH)
        def _():
            _fire_dst(k + 1, dn)

        @pl.when(k > 1)
        def _():
            pltpu.make_async_copy(
                lobuf.at[d], idx_hbm.at[0, pl.ds(0, 8)], wsem).wait()
            pltpu.make_async_copy(
                hibuf.at[d], idx_hbm.at[1, pl.ds(0, 8)], wsem).wait()

        for j2 in range(8):
            for kk in range(8):
                sl = pl.ds(kk * 16, 16)
                dv = dstbuf[d, j2, sl]
                trash = HALF + (dv & 127)
                lobuf[d, j2, sl] = jnp.where(dv < HALF, dv, trash)
                inhi = (dv >= HALF) & (dv < NHI)
                hibuf[d, j2, sl] = jnp.where(inhi, dv - HALF, trash)

        for j2 in range(8):
            pltpu.async_copy(onesbuf, degacc.at[dstbuf.at[d, j2]], degsem,
                             add=True)
        pltpu.async_copy(lobuf.at[d],
                         idx_hbm.at[0, pl.ds(erow0 + k * 8, 8)], wsem)
        pltpu.async_copy(hibuf.at[d],
                         idx_hbm.at[1, pl.ds(erow0 + k * 8, 8)], wsem)

    for j2 in range(8):
        pltpu.make_async_copy(
            onesbuf, degacc.at[dstbuf.at[0, j2]], degsem).wait()
    for _ in range(2):
        pltpu.make_async_copy(
            lobuf.at[0], idx_hbm.at[0, pl.ds(0, 8)], wsem).wait()
        pltpu.make_async_copy(
            hibuf.at[0], idx_hbm.at[1, pl.ds(0, 8)], wsem).wait()

    plsc.subcore_barrier()

    pltpu.sync_copy(degacc.at[pl.ds(s * (N_PAD // NS), N_PAD // NS)],
                    degpart_hbm.at[c, pl.ds(s * (N_PAD // NS), N_PAD // NS)])


def _p1(sid2d, col2d, dst2d, ctable):
    f = pl.kernel(
        _p1_body,
        out_type=[
            jax.ShapeDtypeStruct((N_PAD, HIDDEN), jnp.float32),
            jax.ShapeDtypeStruct((NC, N_PAD), jnp.float32),
            jax.ShapeDtypeStruct((NC, ER, 128), jnp.int32),
        ],
        mesh=_mesh,
        compiler_params=pltpu.CompilerParams(use_tc_tiling_on_sc=False),
        scratch_types=[
            pltpu.VMEM_SHARED((DEGROWS,), jnp.float32),
            pltpu.VMEM((2, 128), jnp.int32),
            pltpu.VMEM((2, 128), jnp.int32),
            pltpu.VMEM((2, 128), jnp.int32),
            pltpu.VMEM((2, 128, HIDDEN), jnp.float32),
            pltpu.VMEM((2, 8, 128), jnp.int32),
            pltpu.VMEM((2, 8, 128), jnp.int32),
            pltpu.VMEM((2, 8, 128), jnp.int32),
            pltpu.VMEM((128,), jnp.float32),
            pltpu.VMEM((DEG_STRIPE,), jnp.float32),
            pltpu.SemaphoreType.DMA,
            pltpu.SemaphoreType.DMA,
            pltpu.SemaphoreType.DMA,
            pltpu.SemaphoreType.DMA,
            pltpu.SemaphoreType.DMA,
            pltpu.SemaphoreType.DMA,
        ],
    )
    return f(sid2d, col2d, dst2d, ctable)


def _conv_body(hp_hbm, src_hbm, idx_hbm,
               accout_hbm,
               acc, srcidx, sidx, rowbuf,
               isem, gsem, ssem, zsem):
    c = lax.axis_index("c")
    s = lax.axis_index("s")

    zeros16 = jnp.zeros((16,), jnp.float32)

    @pl.loop(0, 128)
    def _initz(i):
        for b in range(CV_K):
            rowbuf[b, i, pl.ds(0, 16)] = zeros16
            rowbuf[b, i, pl.ds(16, 16)] = zeros16

    for z in range(24):
        pltpu.async_copy(
            rowbuf.at[0], acc.at[pl.ds(s * ACC_STRIPE + z * 128, 128)], zsem)
    pltpu.async_copy(rowbuf.at[0, pl.ds(0, 64)],
                     acc.at[pl.ds(s * ACC_STRIPE + 24 * 128, 64)], zsem)
    for z in range(24):
        pltpu.make_async_copy(rowbuf.at[0], acc.at[pl.ds(0, 128)], zsem).wait()
    pltpu.make_async_copy(rowbuf.at[0, pl.ds(0, 64)],
                          acc.at[pl.ds(0, 64)], zsem).wait()
    plsc.subcore_barrier()

    row0 = s * CV_ROWS

    def _fire_idx(sup, d):
        for b in range(CV_K):
            r = row0 + sup * CV_K + b
            pltpu.async_copy(src_hbm.at[r], srcidx.at[d, b], isem)
            pltpu.async_copy(idx_hbm.at[c, r], sidx.at[d, b], isem)

    _fire_idx(0, 0)

    @pl.loop(0, CV_SUP)
    def _super(sp):
        d = lax.rem(sp, 2)
        dn = lax.rem(sp + 1, 2)
        @pl.when(sp > 0)
        def _():
            for b in range(CV_K):
                pltpu.make_async_copy(
                    rowbuf.at[b], acc.at[sidx.at[dn, b]], ssem).wait()
        for b in range(CV_K):
            pltpu.make_async_copy(src_hbm.at[0], srcidx.at[d, b], isem).wait()
            pltpu.make_async_copy(src_hbm.at[0], sidx.at[d, b], isem).wait()
        gd = [pltpu.async_copy(hp_hbm.at[srcidx.at[d, b]], rowbuf.at[b],
                               gsem) for b in range(CV_K)]
        @pl.when(sp + 1 < CV_SUP)
        def _():
            _fire_idx(sp + 1, dn)
        for g in gd:
            g.wait()
        for b in range(CV_K):
            pltpu.async_copy(rowbuf.at[b], acc.at[sidx.at[d, b]], ssem,
                             add=True)

    for b in range(CV_K):
        pltpu.make_async_copy(
            rowbuf.at[b], acc.at[sidx.at[0, b]], ssem).wait()

    plsc.subcore_barrier()
    pltpu.sync_copy(
        acc.at[pl.ds(s * WB_STRIPE, WB_STRIPE)],
        accout_hbm.at[pl.ds(c * HALF + s * WB_STRIPE, WB_STRIPE)])


def _conv(hp, src2d, idxarr):
    f = pl.kernel(
        _conv_body,
        out_type=jax.ShapeDtypeStruct((N_PAD, HIDDEN), jnp.float32),
        mesh=_mesh,
        compiler_params=pltpu.CompilerParams(use_tc_tiling_on_sc=False),
        scratch_types=[
            pltpu.VMEM_SHARED((ACCROWS, HIDDEN), jnp.float32),
            pltpu.VMEM((2, CV_K, 128), jnp.int32),
            pltpu.VMEM((2, CV_K, 128), jnp.int32),
            pltpu.VMEM((CV_K, 128, HIDDEN), jnp.float32),
            pltpu.SemaphoreType.DMA,
            pltpu.SemaphoreType.DMA,
            pltpu.SemaphoreType.DMA,
            pltpu.SemaphoreType.DMA,
        ],
    )
    return f(hp, src2d, idxarr)


def _scale1_body(d0_ref, d1_ref, h1_ref, dinv_ref, h1p_ref):
    deg = d0_ref[0, 0] + d1_ref[0, 0] + 1.0
    dv = lax.rsqrt(deg)
    dinv_ref[0, 0] = dv
    h1p_ref[0] = h1_ref[0] * dv[:, None]


def _scale1(d0, d1, h1):
    return pl.pallas_call(
        _scale1_body,
        grid=(NB,),
        in_specs=[
            pl.BlockSpec((1, 1, RB), lambda i: (i, 0, 0)),
            pl.BlockSpec((1, 1, RB), lambda i: (i, 0, 0)),
            pl.BlockSpec((1, RB, HIDDEN), lambda i: (i, 0, 0)),
        ],
        out_specs=[
            pl.BlockSpec((1, 1, RB), lambda i: (i, 0, 0)),
            pl.BlockSpec((1, RB, HIDDEN), lambda i: (i, 0, 0)),
        ],
        out_shape=[
            jax.ShapeDtypeStruct((NB, 1, RB), jnp.float32),
            jax.ShapeDtypeStruct((NB, RB, HIDDEN), jnp.float32),
        ],
    )(d0, d1, h1)


def _scale2_body(acc_ref, h1p_ref, dinv_ref, b1_ref, w2_ref, h2p_ref):
    dv = dinv_ref[0, 0]
    out1 = jnp.maximum((acc_ref[0] + h1p_ref[0]) * dv[:, None] + b1_ref[0],
                       0.0)
    h2 = jnp.dot(out1, w2_ref[...], preferred_element_type=jnp.float32)
    h2p_ref[0] = h2 * dv[:, None]


def _scale2(acc1, h1p, dinv, b1, W2):
    return pl.pallas_call(
        _scale2_body,
        grid=(NB,),
        in_specs=[
            pl.BlockSpec((1, RB, HIDDEN), lambda i: (i, 0, 0)),
            pl.BlockSpec((1, RB, HIDDEN), lambda i: (i, 0, 0)),
            pl.BlockSpec((1, 1, RB), lambda i: (i, 0, 0)),
            pl.BlockSpec((1, HIDDEN), lambda i: (0, 0)),
            pl.BlockSpec((HIDDEN, HIDDEN), lambda i: (0, 0)),
        ],
        out_specs=pl.BlockSpec((1, RB, HIDDEN), lambda i: (i, 0, 0)),
        out_shape=jax.ShapeDtypeStruct((NB, RB, HIDDEN), jnp.float32),
    )(acc1, h1p, dinv, b1, W2)


def _pool_body(acc_ref, h2p_ref, dinv_ref, batch_ref, b2_ref, wc_ref, bc_ref,
               out_ref, sums, cnt):
    i = pl.program_id(0)

    @pl.when(i == 0)
    def _():
        sums[...] = jnp.zeros((B, HIDDEN), jnp.float32)
        cnt[...] = jnp.zeros((B, 1), jnp.float32)

    dv = dinv_ref[0, 0]
    out2 = jnp.maximum((acc_ref[0] + h2p_ref[0]) * dv[:, None] + b2_ref[0],
                       0.0)
    bt = batch_ref[0, 0]
    out2 = jnp.where(bt[:, None] >= 0, out2, 0.0)
    oh = (lax.broadcasted_iota(jnp.int32, (B, RB), 0)
          == bt[None, :]).astype(jnp.float32)
    sums[...] += jnp.dot(oh, out2, preferred_element_type=jnp.float32)
    cnt[...] += jnp.sum(oh, axis=1, keepdims=True)

    @pl.when(i == NB - 1)
    def _():
        pooled = sums[...] / jnp.maximum(cnt[...], 1.0)
        out_ref[...] = (jnp.dot(pooled, wc_ref[...],
                                preferred_element_type=jnp.float32)
                        + bc_ref[0])


def _pool(acc2, h2p, dinv, batch2d, b2, Wc, bc):
    return pl.pallas_call(
        _pool_body,
        grid=(NB,),
        in_specs=[
            pl.BlockSpec((1, RB, HIDDEN), lambda i: (i, 0, 0)),
            pl.BlockSpec((1, RB, HIDDEN), lambda i: (i, 0, 0)),
            pl.BlockSpec((1, 1, RB), lambda i: (i, 0, 0)),
            pl.BlockSpec((1, 1, RB), lambda i: (i, 0, 0)),
            pl.BlockSpec((1, HIDDEN), lambda i: (0, 0)),
            pl.BlockSpec((HIDDEN, NUM_CLASSES), lambda i: (0, 0)),
            pl.BlockSpec((1, NUM_CLASSES), lambda i: (0, 0)),
        ],
        out_specs=pl.BlockSpec((B, NUM_CLASSES), lambda i: (0, 0)),
        out_shape=jax.ShapeDtypeStruct((B, NUM_CLASSES), jnp.float32),
        scratch_shapes=[
            pltpu.VMEM((B, HIDDEN), jnp.float32),
            pltpu.VMEM((B, 1), jnp.float32),
        ],
    )(acc2, h2p, dinv, batch2d, b2, Wc, bc)


def kernel(shape_ids, color_ids, edge_index, batch, shape_emb, color_emb,
           W1, b1, W2, b2, Wc, bc):
    i32 = jnp.int32
    sid2d = jnp.pad(shape_ids.astype(i32), (0, N_PAD - N)).reshape(N_PAD // 128, 128)
    col2d = jnp.pad(color_ids.astype(i32), (0, N_PAD - N)).reshape(N_PAD // 128, 128)
    src2d = jnp.pad(edge_index[0].astype(i32), (0, E_PAD - E)).reshape(ER, 128)
    dst2d = jnp.pad(edge_index[1].astype(i32), (0, E_PAD - E),
                    constant_values=N_PAD).reshape(ER, 128)
    batch2d = jnp.pad(batch.astype(i32), (0, N_PAD - N),
                      constant_values=-1).reshape(NB, 1, RB)

    ctable = _prep(shape_emb, color_emb, W1)
    h1, degpart, idxarr = _p1(sid2d, col2d, dst2d, ctable)

    d0 = degpart[0].reshape(NB, 1, RB)
    d1 = degpart[1].reshape(NB, 1, RB)
    dinv, h1p3 = _scale1(d0, d1, h1.reshape(NB, RB, HIDDEN))
    h1p = h1p3.reshape(N_PAD, HIDDEN)

    acc1 = _conv(h1p, src2d, idxarr)
    h2p3 = _scale2(acc1.reshape(NB, RB, HIDDEN), h1p3, dinv,
                   b1.reshape(1, HIDDEN), W2)
    acc2 = _conv(h2p3.reshape(N_PAD, HIDDEN), src2d, idxarr)

    return _pool(acc2.reshape(NB, RB, HIDDEN), h2p3, dinv, batch2d,
                 b2.reshape(1, HIDDEN), Wc, bc.reshape(1, NUM_CLASSES))


# speedup vs baseline: 29.1607x; 1.0080x over previous
import functools

import jax
import jax.numpy as jnp
from jax import lax
from jax.experimental import pallas as pl
from jax.experimental.pallas import tpu as pltpu
from jax.experimental.pallas import tpu_sc as plsc

N = 100000
E = 1600000
B = 128
HIDDEN = 32
NUM_CLASSES = 2

NC, NS = 2, 16
NW = NC * NS
N_PAD = 102400
NODES_PER_TILE = N_PAD // NW
NROWCH = NODES_PER_TILE // 128
E_PAD = 1605632
ER = E_PAD // 128
P1_ROWS = ER // NW
P1_CH = P1_ROWS // 8
CV_ROWS = ER // NS
CV_K = 2
CV_SUP = CV_ROWS // CV_K
HALF = 50048
NHI = 2 * HALF
ACCROWS = 50176
ACC_STRIPE = ACCROWS // NS
WB_STRIPE = HALF // NS
DEGROWS = 104448
DEG_STRIPE = DEGROWS // NS
RB = 4096
NB = N_PAD // RB

_mesh = plsc.VectorSubcoreMesh(core_axis_name="c", subcore_axis_name="s")


def _prep_body(se_ref, ce_ref, w1_ref, ct_ref):
    a = jnp.dot(se_ref[...], w1_ref[0:8, :], preferred_element_type=jnp.float32)
    b = jnp.dot(ce_ref[...], w1_ref[8:12, :], preferred_element_type=jnp.float32)
    row = lax.broadcasted_iota(jnp.int32, (260, 26), 0)
    ohs = (row // 10 == lax.broadcasted_iota(jnp.int32, (260, 26), 1)).astype(jnp.float32)
    rowc = lax.broadcasted_iota(jnp.int32, (260, 10), 0)
    ohc = (rowc % 10 == lax.broadcasted_iota(jnp.int32, (260, 10), 1)).astype(jnp.float32)
    ct_ref[...] = (jnp.dot(ohs, a, preferred_element_type=jnp.float32)
                   + jnp.dot(ohc, b, preferred_element_type=jnp.float32))


def _prep(shape_emb, color_emb, W1):
    return pl.pallas_call(
        _prep_body,
        out_shape=jax.ShapeDtypeStruct((260, HIDDEN), jnp.float32),
    )(shape_emb, color_emb, W1)


def _p1_body(sid_hbm, col_hbm, dst_hbm, ct_hbm,
             h1_hbm, degpart_hbm, idx_hbm,
             degacc, sbuf, cbuf, cidbuf, rowbuf, dstbuf, lobuf, hibuf,
             onesbuf, zbuf,
             nsem, ngsem, hwsem, dsem, degsem, wsem):
    c = lax.axis_index("c")
    s = lax.axis_index("s")
    w = c * NS + s

    zeros16 = jnp.zeros((16,), jnp.float32)
    ones16 = jnp.ones((16,), jnp.float32)

    @pl.loop(0, 8)
    def _init(k):
        onesbuf[pl.ds(k * 16, 16)] = ones16

    @pl.loop(0, DEG_STRIPE // 16)
    def _initz(k):
        zbuf[pl.ds(k * 16, 16)] = zeros16

    pltpu.sync_copy(zbuf, degacc.at[pl.ds(s * DEG_STRIPE, DEG_STRIPE)])

    nrow0 = w * NROWCH

    def _fire_idcopy(j, d):
        pltpu.async_copy(sid_hbm.at[nrow0 + j], sbuf.at[d], nsem)
        pltpu.async_copy(col_hbm.at[nrow0 + j], cbuf.at[d], nsem)

    _fire_idcopy(0, 0)

    @pl.loop(0, NROWCH)
    def _node_chunk(j):
        d = lax.rem(j, 2)
        dn = lax.rem(j + 1, 2)
        pltpu.make_async_copy(sid_hbm.at[nrow0 + j], sbuf.at[d], nsem).wait()
        pltpu.make_async_copy(col_hbm.at[nrow0 + j], cbuf.at[d], nsem).wait()

        @pl.when(j + 1 < NROWCH)
        def _():
            _fire_idcopy(j + 1, dn)

        @pl.when(j > 1)
        def _():
            pltpu.make_async_copy(
                rowbuf.at[d], h1_hbm.at[pl.ds(0, 128)], hwsem).wait()

        for k in range(8):
            sl = pl.ds(k * 16, 16)
            cidbuf[d, sl] = sbuf[d, sl] * 10 + cbuf[d, sl]
        pltpu.async_copy(ct_hbm.at[cidbuf.at[d]], rowbuf.at[d], ngsem).wait()
        pltpu.async_copy(
            rowbuf.at[d],
            h1_hbm.at[pl.ds(w * NODES_PER_TILE + j * 128, 128)], hwsem)

    pltpu.make_async_copy(rowbuf.at[0], h1_hbm.at[pl.ds(0, 128)], hwsem).wait()
    pltpu.make_async_copy(rowbuf.at[0], h1_hbm.at[pl.ds(0, 128)], hwsem).wait()

    plsc.subcore_barrier()

    erow0 = w * P1_ROWS

    def _fire_dst(k, d):
        pltpu.async_copy(dst_hbm.at[pl.ds(erow0 + k * 8, 8)], dstbuf.at[d], dsem)

    _fire_dst(0, 0)

    @pl.loop(0, P1_CH)
    def _edge_chunk(k):
        d = lax.rem(k, 2)
        dn = lax.rem(k + 1, 2)
        pltpu.make_async_copy(
            dst_hbm.at[pl.ds(erow0, 8)], dstbuf.at[d], dsem).wait()

        @pl.when(k > 0)
        def _():
            for j2 in range(8):
                pltpu.make_async_copy(
                    onesbuf, degacc.at[dstbuf.at[dn, j2]], degsem).wait()

        @pl.when(k + 1 < P1_CH)
        def _():
            _fire_dst(k + 1, dn)

        @pl.when(k > 1)
        def _():
            pltpu.make_async_copy(
                lobuf.at[d], idx_hbm.at[0, pl.ds(0, 8)], wsem).wait()
            pltpu.make_async_copy(
                hibuf.at[d], idx_hbm.at[1, pl.ds(0, 8)], wsem).wait()

        for j2 in range(8):
            for kk in range(8):
                sl = pl.ds(kk * 16, 16)
                dv = dstbuf[d, j2, sl]
                trash = HALF + (dv & 127)
                lobuf[d, j2, sl] = jnp.where(dv < HALF, dv, trash)
                inhi = (dv >= HALF) & (dv < NHI)
                hibuf[d, j2, sl] = jnp.where(inhi, dv - HALF, trash)

        for j2 in range(8):
            pltpu.async_copy(onesbuf, degacc.at[dstbuf.at[d, j2]], degsem,
                             add=True)
        pltpu.async_copy(lobuf.at[d],
                         idx_hbm.at[0, pl.ds(erow0 + k * 8, 8)], wsem)
        pltpu.async_copy(hibuf.at[d],
                         idx_hbm.at[1, pl.ds(erow0 + k * 8, 8)], wsem)

    for j2 in range(8):
        pltpu.make_async_copy(
            onesbuf, degacc.at[dstbuf.at[0, j2]], degsem).wait()
    for _ in range(2):
        pltpu.make_async_copy(
            lobuf.at[0], idx_hbm.at[0, pl.ds(0, 8)], wsem).wait()
        pltpu.make_async_copy(
            hibuf.at[0], idx_hbm.at[1, pl.ds(0, 8)], wsem).wait()

    plsc.subcore_barrier()

    pltpu.sync_copy(degacc.at[pl.ds(s * (N_PAD // NS), N_PAD // NS)],
                    degpart_hbm.at[c, pl.ds(s * (N_PAD // NS), N_PAD // NS)])


def _p1(sid2d, col2d, dst2d, ctable):
    f = pl.kernel(
        _p1_body,
        out_type=[
            jax.ShapeDtypeStruct((N_PAD, HIDDEN), jnp.float32),
            jax.ShapeDtypeStruct((NC, N_PAD), jnp.float32),
            jax.ShapeDtypeStruct((NC, ER, 128), jnp.int32),
        ],
        mesh=_mesh,
        compiler_params=pltpu.CompilerParams(use_tc_tiling_on_sc=False),
        scratch_types=[
            pltpu.VMEM_SHARED((DEGROWS,), jnp.float32),
            pltpu.VMEM((2, 128), jnp.int32),
            pltpu.VMEM((2, 128), jnp.int32),
            pltpu.VMEM((2, 128), jnp.int32),
            pltpu.VMEM((2, 128, HIDDEN), jnp.float32),
            pltpu.VMEM((2, 8, 128), jnp.int32),
            pltpu.VMEM((2, 8, 128), jnp.int32),
            pltpu.VMEM((2, 8, 128), jnp.int32),
            pltpu.VMEM((128,), jnp.float32),
            pltpu.VMEM((DEG_STRIPE,), jnp.float32),
            pltpu.SemaphoreType.DMA,
            pltpu.SemaphoreType.DMA,
            pltpu.SemaphoreType.DMA,
            pltpu.SemaphoreType.DMA,
            pltpu.SemaphoreType.DMA,
            pltpu.SemaphoreType.DMA,
        ],
    )
    return f(sid2d, col2d, dst2d, ctable)


def _conv_body(hp_hbm, src_hbm, idx_hbm,
               accout_hbm,
               acc, srcidx, sidx, rowbuf,
               isem, gsem, ssem, zsem):
    c = lax.axis_index("c")
    s = lax.axis_index("s")

    zeros16 = jnp.zeros((16,), jnp.float32)

    @pl.loop(0, 128)
    def _initz(i):
        for d in range(2):
            for b in range(CV_K):
                rowbuf[d, b, i, pl.ds(0, 16)] = zeros16
                rowbuf[d, b, i, pl.ds(16, 16)] = zeros16

    for z in range(24):
        pltpu.async_copy(
            rowbuf.at[0, 0], acc.at[pl.ds(s * ACC_STRIPE + z * 128, 128)], zsem)
    pltpu.async_copy(rowbuf.at[0, 0, pl.ds(0, 64)],
                     acc.at[pl.ds(s * ACC_STRIPE + 24 * 128, 64)], zsem)
    for z in range(24):
        pltpu.make_async_copy(rowbuf.at[0, 0], acc.at[pl.ds(0, 128)],
                              zsem).wait()
    pltpu.make_async_copy(rowbuf.at[0, 0, pl.ds(0, 64)],
                          acc.at[pl.ds(0, 64)], zsem).wait()
    plsc.subcore_barrier()

    row0 = s * CV_ROWS

    def _fire_idx(sup, d):
        for b in range(CV_K):
            r = row0 + sup * CV_K + b
            pltpu.async_copy(src_hbm.at[r], srcidx.at[d, b], isem)
            pltpu.async_copy(idx_hbm.at[c, r], sidx.at[d, b], isem)

    _fire_idx(0, 0)

    @pl.loop(0, CV_SUP)
    def _super(sp):
        d = lax.rem(sp, 2)
        dn = lax.rem(sp + 1, 2)
        for b in range(CV_K):
            pltpu.make_async_copy(src_hbm.at[0], srcidx.at[d, b], isem).wait()
            pltpu.make_async_copy(src_hbm.at[0], sidx.at[d, b], isem).wait()
        gd = [pltpu.async_copy(hp_hbm.at[srcidx.at[d, b]], rowbuf.at[d, b],
                               gsem) for b in range(CV_K)]
        @pl.when(sp > 0)
        def _():
            for b in range(CV_K):
                pltpu.make_async_copy(
                    rowbuf.at[dn, b], acc.at[sidx.at[dn, b]], ssem).wait()
        @pl.when(sp + 1 < CV_SUP)
        def _():
            _fire_idx(sp + 1, dn)
        for g in gd:
            g.wait()
        for b in range(CV_K):
            pltpu.async_copy(rowbuf.at[d, b], acc.at[sidx.at[d, b]], ssem,
                             add=True)

    for b in range(CV_K):
        pltpu.make_async_copy(
            rowbuf.at[1, b], acc.at[sidx.at[1, b]], ssem).wait()

    plsc.subcore_barrier()
    pltpu.sync_copy(
        acc.at[pl.ds(s * WB_STRIPE, WB_STRIPE)],
        accout_hbm.at[pl.ds(c * HALF + s * WB_STRIPE, WB_STRIPE)])


def _conv(hp, src2d, idxarr):
    f = pl.kernel(
        _conv_body,
        out_type=jax.ShapeDtypeStruct((N_PAD, HIDDEN), jnp.float32),
        mesh=_mesh,
        compiler_params=pltpu.CompilerParams(use_tc_tiling_on_sc=False),
        scratch_types=[
            pltpu.VMEM_SHARED((ACCROWS, HIDDEN), jnp.float32),
            pltpu.VMEM((2, CV_K, 128), jnp.int32),
            pltpu.VMEM((2, CV_K, 128), jnp.int32),
            pltpu.VMEM((2, CV_K, 128, HIDDEN), jnp.float32),
            pltpu.SemaphoreType.DMA,
            pltpu.SemaphoreType.DMA,
            pltpu.SemaphoreType.DMA,
            pltpu.SemaphoreType.DMA,
        ],
    )
    return f(hp, src2d, idxarr)


def _scale1_body(d0_ref, d1_ref, h1_ref, dinv_ref, h1p_ref):
    deg = d0_ref[0, 0] + d1_ref[0, 0] + 1.0
    dv = lax.rsqrt(deg)
    dinv_ref[0, 0] = dv
    h1p_ref[0] = h1_ref[0] * dv[:, None]


def _scale1(d0, d1, h1):
    return pl.pallas_call(
        _scale1_body,
        grid=(NB,),
        in_specs=[
            pl.BlockSpec((1, 1, RB), lambda i: (i, 0, 0)),
            pl.BlockSpec((1, 1, RB), lambda i: (i, 0, 0)),
            pl.BlockSpec((1, RB, HIDDEN), lambda i: (i, 0, 0)),
        ],
        out_specs=[
            pl.BlockSpec((1, 1, RB), lambda i: (i, 0, 0)),
            pl.BlockSpec((1, RB, HIDDEN), lambda i: (i, 0, 0)),
        ],
        out_shape=[
            jax.ShapeDtypeStruct((NB, 1, RB), jnp.float32),
            jax.ShapeDtypeStruct((NB, RB, HIDDEN), jnp.float32),
        ],
    )(d0, d1, h1)


def _scale2_body(acc_ref, h1p_ref, dinv_ref, b1_ref, w2_ref, h2p_ref):
    dv = dinv_ref[0, 0]
    out1 = jnp.maximum((acc_ref[0] + h1p_ref[0]) * dv[:, None] + b1_ref[0],
                       0.0)
    h2 = jnp.dot(out1, w2_ref[...], preferred_element_type=jnp.float32)
    h2p_ref[0] = h2 * dv[:, None]


def _scale2(acc1, h1p, dinv, b1, W2):
    return pl.pallas_call(
        _scale2_body,
        grid=(NB,),
        in_specs=[
            pl.BlockSpec((1, RB, HIDDEN), lambda i: (i, 0, 0)),
            pl.BlockSpec((1, RB, HIDDEN), lambda i: (i, 0, 0)),
            pl.BlockSpec((1, 1, RB), lambda i: (i, 0, 0)),
            pl.BlockSpec((1, HIDDEN), lambda i: (0, 0)),
            pl.BlockSpec((HIDDEN, HIDDEN), lambda i: (0, 0)),
        ],
        out_specs=pl.BlockSpec((1, RB, HIDDEN), lambda i: (i, 0, 0)),
        out_shape=jax.ShapeDtypeStruct((NB, RB, HIDDEN), jnp.float32),
    )(acc1, h1p, dinv, b1, W2)


def _pool_body(acc_ref, h2p_ref, dinv_ref, batch_ref, b2_ref, wc_ref, bc_ref,
               out_ref, sums, cnt):
    i = pl.program_id(0)

    @pl.when(i == 0)
    def _():
        sums[...] = jnp.zeros((B, HIDDEN), jnp.float32)
        cnt[...] = jnp.zeros((B, 1), jnp.float32)

    dv = dinv_ref[0, 0]
    out2 = jnp.maximum((acc_ref[0] + h2p_ref[0]) * dv[:, None] + b2_ref[0],
                       0.0)
    bt = batch_ref[0, 0]
    out2 = jnp.where(bt[:, None] >= 0, out2, 0.0)
    oh = (lax.broadcasted_iota(jnp.int32, (B, RB), 0)
          == bt[None, :]).astype(jnp.float32)
    sums[...] += jnp.dot(oh, out2, preferred_element_type=jnp.float32)
    cnt[...] += jnp.sum(oh, axis=1, keepdims=True)

    @pl.when(i == NB - 1)
    def _():
        pooled = sums[...] / jnp.maximum(cnt[...], 1.0)
        out_ref[...] = (jnp.dot(pooled, wc_ref[...],
                                preferred_element_type=jnp.float32)
                        + bc_ref[0])


def _pool(acc2, h2p, dinv, batch2d, b2, Wc, bc):
    return pl.pallas_call(
        _pool_body,
        grid=(NB,),
        in_specs=[
            pl.BlockSpec((1, RB, HIDDEN), lambda i: (i, 0, 0)),
            pl.BlockSpec((1, RB, HIDDEN), lambda i: (i, 0, 0)),
            pl.BlockSpec((1, 1, RB), lambda i: (i, 0, 0)),
            pl.BlockSpec((1, 1, RB), lambda i: (i, 0, 0)),
            pl.BlockSpec((1, HIDDEN), lambda i: (0, 0)),
            pl.BlockSpec((HIDDEN, NUM_CLASSES), lambda i: (0, 0)),
            pl.BlockSpec((1, NUM_CLASSES), lambda i: (0, 0)),
        ],
        out_specs=pl.BlockSpec((B, NUM_CLASSES), lambda i: (0, 0)),
        out_shape=jax.ShapeDtypeStruct((B, NUM_CLASSES), jnp.float32),
        scratch_shapes=[
            pltpu.VMEM((B, HIDDEN), jnp.float32),
            pltpu.VMEM((B, 1), jnp.float32),
        ],
    )(acc2, h2p, dinv, batch2d, b2, Wc, bc)


def kernel(shape_ids, color_ids, edge_index, batch, shape_emb, color_emb,
           W1, b1, W2, b2, Wc, bc):
    i32 = jnp.int32
    sid2d = jnp.pad(shape_ids.astype(i32), (0, N_PAD - N)).reshape(N_PAD // 128, 128)
    col2d = jnp.pad(color_ids.astype(i32), (0, N_PAD - N)).reshape(N_PAD // 128, 128)
    src2d = jnp.pad(edge_index[0].astype(i32), (0, E_PAD - E)).reshape(ER, 128)
    dst2d = jnp.pad(edge_index[1].astype(i32), (0, E_PAD - E),
                    constant_values=N_PAD).reshape(ER, 128)
    batch2d = jnp.pad(batch.astype(i32), (0, N_PAD - N),
                      constant_values=-1).reshape(NB, 1, RB)

    ctable = _prep(shape_emb, color_emb, W1)
    h1, degpart, idxarr = _p1(sid2d, col2d, dst2d, ctable)

    d0 = degpart[0].reshape(NB, 1, RB)
    d1 = degpart[1].reshape(NB, 1, RB)
    dinv, h1p3 = _scale1(d0, d1, h1.reshape(NB, RB, HIDDEN))
    h1p = h1p3.reshape(N_PAD, HIDDEN)

    acc1 = _conv(h1p, src2d, idxarr)
    h2p3 = _scale2(acc1.reshape(NB, RB, HIDDEN), h1p3, dinv,
                   b1.reshape(1, HIDDEN), W2)
    acc2 = _conv(h2p3.reshape(N_PAD, HIDDEN), src2d, idxarr)

    return _pool(acc2.reshape(NB, RB, HIDDEN), h2p3, dinv, batch2d,
                 b2.reshape(1, HIDDEN), Wc, bc.reshape(1, NUM_CLASSES))
